# baseline (speedup 1.0000x reference)
import functools

import jax
import jax.numpy as jnp
from jax import lax
from jax.experimental import pallas as pl
from jax.experimental.pallas import tpu as pltpu

N_USERS = 5000
N_ITEMS = 3000
N_ENT = 5000
N_NODES = 10000
CKG_REL = 16
UKG_REL = 4
D = 128
HOPS = 2
IMG_D = 512
TXT_D = 768
B = 4096


def _mlp_body(x_ref, w1_ref, b1_ref, w2_ref, b2_ref, o_ref):
    h = jnp.dot(x_ref[...], w1_ref[...], preferred_element_type=jnp.float32)
    h = h + b1_ref[...]
    h = jnp.where(h >= 0, h, 0.01 * h)
    o = jnp.dot(h, w2_ref[...], preferred_element_type=jnp.float32)
    o_ref[...] = o + b2_ref[...]


def _mlp(x, w1, b1, w2, b2):
    n = x.shape[0]
    return pl.pallas_call(
        _mlp_body,
        out_shape=jax.ShapeDtypeStruct((n, D), jnp.float32),
        grid=(3,),
        in_specs=[
            pl.BlockSpec((n // 3, x.shape[1]), lambda i: (i, 0)),
            pl.BlockSpec(w1.shape, lambda i: (0, 0)),
            pl.BlockSpec(b1.shape, lambda i: (0,)),
            pl.BlockSpec(w2.shape, lambda i: (0, 0)),
            pl.BlockSpec(b2.shape, lambda i: (0,)),
        ],
        out_specs=pl.BlockSpec((n // 3, D), lambda i: (i, 0)),
    )(x, w1, b1, w2, b2)


def _graph_conv(ego, head, tail, edge_type, rel_emb, n_hops):
    n = ego.shape[0]
    deg = jax.ops.segment_sum(jnp.ones(head.shape[0], dtype=ego.dtype), head,
                              num_segments=n)
    deg = jnp.clip(deg, 1.0, None)[:, None]
    out = ego
    cur = ego
    for _ in range(n_hops):
        msg = cur[tail] * rel_emb[edge_type]
        cur = jax.ops.segment_sum(msg, head, num_segments=n) / deg
        out = out + cur
    return out / float(n_hops + 1)


def kernel(user_ids, item_ids, ckg_edge_index, ckg_edge_type, ukg_edge_index, ukg_edge_type, image_features, text_features, other_emb_image, other_emb_text, ckg_rel_image, ckg_rel_text, ukg_rel_image, ukg_rel_text, W_img1, b_img1, W_img2, b_img2, W_txt1, b_txt1, W_txt2, b_txt2):
    img_feat = _mlp(image_features, W_img1, b_img1, W_img2, b_img2)
    txt_feat = _mlp(text_features, W_txt1, b_txt1, W_txt2, b_txt2)

    ego = jnp.concatenate([
        jnp.concatenate([img_feat, other_emb_image], axis=0),
        jnp.concatenate([txt_feat, other_emb_text], axis=0),
    ], axis=1)
    ckg_rel = jnp.concatenate([ckg_rel_image, ckg_rel_text], axis=1)
    ukg_rel = jnp.concatenate([ukg_rel_image, ukg_rel_text], axis=1)

    all_emb = _graph_conv(ego, ckg_edge_index[0], ckg_edge_index[1],
                          ckg_edge_type, ckg_rel, HOPS)
    fu = _graph_conv(all_emb[N_ENT:], ukg_edge_index[0], ukg_edge_index[1],
                     ukg_edge_type, ukg_rel, HOPS)

    uidx = jnp.mod(user_ids - N_ENT, N_USERS)
    user_embed = fu[uidx]
    item_embed = all_emb[item_ids]
    return jax.nn.sigmoid(jnp.sum(user_embed * item_embed, axis=-1))


# speedup vs baseline: 1.2001x; 1.2001x over previous
import functools

import jax
import jax.numpy as jnp
from jax import lax
from jax.experimental import pallas as pl
from jax.experimental.pallas import tpu as pltpu
from jax.experimental.pallas import tpu_sc as plsc

N_USERS = 5000
N_ITEMS = 3000
N_ENT = 5000
N_NODES = 10000
CKG_REL = 16
UKG_REL = 4
D = 128
HOPS = 2
IMG_D = 512
TXT_D = 768
B = 4096
E_CKG = 320000
E_UKG = 160000

NPAD = 10240
NUPAD = 5120
C = 40
NT = 16
ROWS_T = NPAD // NT
ROWS_U = NUPAD // NT
R = 32
ET_C = E_CKG // NT
ET_U = E_UKG // NT
NCH_C = ET_C // C
NCH_U = ET_U // C
NMOD = 2

_mesh = plsc.VectorSubcoreMesh(core_axis_name="c", subcore_axis_name="s")


def _mlp_body(x_ref, w1_ref, b1_ref, w2_ref, b2_ref, o_ref):
    h = jnp.dot(x_ref[...], w1_ref[...], preferred_element_type=jnp.float32)
    h = h + b1_ref[...]
    h = jnp.where(h >= 0, h, 0.01 * h)
    o = jnp.dot(h, w2_ref[...], preferred_element_type=jnp.float32)
    o_ref[...] = o + b2_ref[...]


def _mlp(x, w1, b1, w2, b2):
    n = x.shape[0]
    return pl.pallas_call(
        _mlp_body,
        out_shape=jax.ShapeDtypeStruct((n, D), jnp.float32),
        grid=(3,),
        in_specs=[
            pl.BlockSpec((n // 3, x.shape[1]), lambda i: (i, 0)),
            pl.BlockSpec(w1.shape, lambda i: (0, 0)),
            pl.BlockSpec(b1.shape, lambda i: (0,)),
            pl.BlockSpec(w2.shape, lambda i: (0, 0)),
            pl.BlockSpec(b2.shape, lambda i: (0,)),
        ],
        out_specs=pl.BlockSpec((n // 3, D), lambda i: (i, 0)),
    )(x, w1, b1, w2, b2)


_CONV_SCRATCH = [
    pltpu.VMEM((C,), jnp.int32),
    pltpu.VMEM((C,), jnp.int32),
    pltpu.VMEM((C,), jnp.int32),
    pltpu.VMEM((C, D), jnp.float32),
    pltpu.VMEM((C, D), jnp.float32),
    pltpu.VMEM((R, D), jnp.float32),
    pltpu.VMEM((R, D), jnp.float32),
    pltpu.VMEM((R, D), jnp.float32),
    pltpu.VMEM((R, D), jnp.float32),
    pltpu.VMEM_SHARED((NPAD, D), jnp.float32),
    pltpu.SemaphoreType.DMA,
    pltpu.SemaphoreType.DMA,
]


def _zero_acc_rows(sid, nsub, tile_rows, accb_v, acc_sh):
    zero16 = jnp.zeros((16,), jnp.float32)

    def zf(r, _):
        for k in range(8):
            accb_v[r, pl.ds(k * 16, 16)] = zero16
        return 0
    lax.fori_loop(0, R, zf, 0)

    def f(s, _):
        rb = sid * tile_rows + s * R
        pltpu.sync_copy(accb_v, acc_sh.at[pl.ds(rb, R), :])
        return 0
    lax.fori_loop(0, nsub, f, 0)


def _make_deg_kernel(out_rows, tile_rows, et, nch):
    nsub = tile_rows // R

    @functools.partial(
        pl.kernel, mesh=_mesh,
        out_type=jax.ShapeDtypeStruct((NMOD * out_rows, D), jnp.float32),
        scratch_types=_CONV_SCRATCH)
    def deg_k(heads_hbm, deg_out, tails_c, types_c, heads_c, rows_v, rel_v,
              accb_v, degb_v, egob_v, cur1b_v, acc_sh, sem_a, sem_b):
        core = lax.axis_index("c")
        sid = lax.axis_index("s")
        one16 = jnp.ones((16,), jnp.float32)

        def of(r, _):
            for k in range(8):
                rows_v[r, pl.ds(k * 16, 16)] = one16
            return 0
        lax.fori_loop(0, C, of, 0)
        _zero_acc_rows(sid, ROWS_T // R, ROWS_T, accb_v, acc_sh)
        plsc.subcore_barrier()
        eoff = sid * et

        def chunk(j, _):
            pltpu.sync_copy(heads_hbm.at[pl.ds(eoff + j * C, C)], heads_c)
            pltpu.sync_copy(rows_v, acc_sh.at[heads_c], add=True)
            return 0
        lax.fori_loop(0, nch, chunk, 0)
        plsc.subcore_barrier()
        base = core * out_rows

        def f(s, _):
            rb = sid * tile_rows + s * R
            pltpu.sync_copy(acc_sh.at[pl.ds(rb, R), :], degb_v)
            pltpu.sync_copy(degb_v, deg_out.at[pl.ds(base + rb, R), :])
            return 0
        lax.fori_loop(0, nsub, f, 0)
    return deg_k


def _make_hop_kernel(out_rows, tile_rows, e_all, et, nch, hop2, ego_off):
    nsub = tile_rows // R

    def body(src_cat, rel_cat, tails2, types2, heads1, deg_hbm, ego_hbm,
             cur1_hbm, dst_out, tails_c, types_c, heads_c, rows_v, rel_v,
             accb_v, degb_v, egob_v, cur1b_v, acc_sh, sem_a, sem_b):
        core = lax.axis_index("c")
        sid = lax.axis_index("s")
        _zero_acc_rows(sid, ROWS_T // R, ROWS_T, accb_v, acc_sh)
        plsc.subcore_barrier()
        eoff = core * e_all + sid * et

        def chunk(j, _):
            coff = eoff + j * C
            pltpu.sync_copy(tails2.at[pl.ds(coff, C)], tails_c)
            pltpu.sync_copy(types2.at[pl.ds(coff, C)], types_c)
            pltpu.sync_copy(heads1.at[pl.ds(coff - core * e_all, C)], heads_c)
            cp1 = pltpu.async_copy(src_cat.at[tails_c], rows_v, sem_a)
            cp2 = pltpu.async_copy(rel_cat.at[types_c], rel_v, sem_b)
            cp1.wait()
            cp2.wait()

            def mul(r, _):
                for k in range(8):
                    sl = pl.ds(k * 16, 16)
                    rows_v[r, sl] = rows_v[r, sl] * rel_v[r, sl]
                return 0
            lax.fori_loop(0, C, mul, 0)
            pltpu.sync_copy(rows_v, acc_sh.at[heads_c], add=True)
            return 0
        lax.fori_loop(0, nch, chunk, 0)
        plsc.subcore_barrier()
        base = core * out_rows

        def sub(s, _):
            rb = sid * tile_rows + s * R
            pltpu.sync_copy(acc_sh.at[pl.ds(rb, R), :], accb_v)
            pltpu.sync_copy(deg_hbm.at[pl.ds(base + rb, R), :], degb_v)
            if hop2:
                pltpu.sync_copy(
                    ego_hbm.at[pl.ds(core * NPAD + ego_off + rb, R), :],
                    egob_v)
                pltpu.sync_copy(cur1_hbm.at[pl.ds(base + rb, R), :], cur1b_v)

            def rowf(r, _):
                rd = 1.0 / jnp.maximum(degb_v[r, pl.ds(0, 16)], 1.0)
                for k in range(8):
                    sl = pl.ds(k * 16, 16)
                    if hop2:
                        v = (egob_v[r, sl] + cur1b_v[r, sl]
                             + accb_v[r, sl] * rd)
                        accb_v[r, sl] = v * (1.0 / 3.0)
                    else:
                        accb_v[r, sl] = accb_v[r, sl] * rd
                return 0
            lax.fori_loop(0, R, rowf, 0)
            pltpu.sync_copy(accb_v, dst_out.at[pl.ds(base + rb, R), :])
            return 0
        lax.fori_loop(0, nsub, sub, 0)

    out_t = jax.ShapeDtypeStruct((NMOD * out_rows, D), jnp.float32)
    if hop2:
        return functools.partial(pl.kernel, mesh=_mesh, out_type=out_t,
                                 scratch_types=_CONV_SCRATCH)(body)

    def body1(src_cat, rel_cat, tails2, types2, heads1, deg_hbm, dst_out,
              *scr):
        return body(src_cat, rel_cat, tails2, types2, heads1, deg_hbm,
                    None, None, dst_out, *scr)
    return functools.partial(pl.kernel, mesh=_mesh, out_type=out_t,
                             scratch_types=_CONV_SCRATCH)(body1)


_deg_ckg = _make_deg_kernel(NPAD, ROWS_T, ET_C, NCH_C)
_deg_ukg = _make_deg_kernel(NUPAD, ROWS_U, ET_U, NCH_U)
_hop1_ckg = _make_hop_kernel(NPAD, ROWS_T, E_CKG, ET_C, NCH_C, False, 0)
_hop2_ckg = _make_hop_kernel(NPAD, ROWS_T, E_CKG, ET_C, NCH_C, True, 0)
_hop1_ukg = _make_hop_kernel(NUPAD, ROWS_U, E_UKG, ET_U, NCH_U, False, 0)
_hop2_ukg = _make_hop_kernel(NUPAD, ROWS_U, E_UKG, ET_U, NCH_U, True, N_ENT)


PW = B // (2 * NT)
PCH = PW // 16


@functools.partial(
    pl.kernel,
    mesh=_mesh,
    out_type=jax.ShapeDtypeStruct((B, 2 * D), jnp.float32),
    scratch_types=[
        pltpu.VMEM((16,), jnp.int32),
        pltpu.VMEM((16,), jnp.int32),
        pltpu.VMEM((16,), jnp.int32),
        pltpu.VMEM((16,), jnp.int32),
        pltpu.VMEM((16, D), jnp.float32),
        pltpu.VMEM((16, D), jnp.float32),
        pltpu.VMEM((16, D), jnp.float32),
        pltpu.VMEM((16, D), jnp.float32),
        pltpu.VMEM((16, 2 * D), jnp.float32),
        pltpu.SemaphoreType.DMA,
    ],
)
def _score_kernel(uidx_hbm, uidxt_hbm, iidx_hbm, iidxt_hbm, fu_cat, all_cat,
                  prod_hbm, uix_img, uix_txt, iix_img, iix_txt,
                  uimg, iimg, utxt, itxt, prodb, sem):
    core = lax.axis_index("c")
    sid = lax.axis_index("s")
    wid = sid * 2 + core
    base = wid * PW

    def chunk(cj, _):
        off = base + cj * 16
        pltpu.sync_copy(uidx_hbm.at[pl.ds(off, 16)], uix_img)
        pltpu.sync_copy(uidxt_hbm.at[pl.ds(off, 16)], uix_txt)
        pltpu.sync_copy(iidx_hbm.at[pl.ds(off, 16)], iix_img)
        pltpu.sync_copy(iidxt_hbm.at[pl.ds(off, 16)], iix_txt)
        c1 = pltpu.async_copy(fu_cat.at[uix_img], uimg, sem)
        c2 = pltpu.async_copy(all_cat.at[iix_img], iimg, sem)
        c3 = pltpu.async_copy(fu_cat.at[uix_txt], utxt, sem)
        c4 = pltpu.async_copy(all_cat.at[iix_txt], itxt, sem)
        c1.wait()
        c2.wait()
        c3.wait()
        c4.wait()

        def pair(p, _):
            for k in range(8):
                sl = pl.ds(k * 16, 16)
                sl2 = pl.ds(D + k * 16, 16)
                prodb[p, sl] = uimg[p, sl] * iimg[p, sl]
                prodb[p, sl2] = utxt[p, sl] * itxt[p, sl]
            return 0
        lax.fori_loop(0, 16, pair, 0)
        pltpu.sync_copy(prodb, prod_hbm.at[pl.ds(off, 16), :])
        return 0
    lax.fori_loop(0, PCH, chunk, 0)


def _score_red_body(x_ref, o_ref):
    s = jnp.sum(x_ref[...], axis=-1)
    o_ref[...] = (1.0 / (1.0 + jnp.exp(-s))).reshape(B // D, D)


def _score_reduce(prod):
    return pl.pallas_call(
        _score_red_body,
        out_shape=jax.ShapeDtypeStruct((B // D, D), jnp.float32),
    )(prod)


def kernel(user_ids, item_ids, ckg_edge_index, ckg_edge_type,
           ukg_edge_index, ukg_edge_type, image_features, text_features,
           other_emb_image, other_emb_text, ckg_rel_image, ckg_rel_text,
           ukg_rel_image, ukg_rel_text, W_img1, b_img1, W_img2, b_img2,
           W_txt1, b_txt1, W_txt2, b_txt2):
    img_feat = _mlp(image_features, W_img1, b_img1, W_img2, b_img2)
    txt_feat = _mlp(text_features, W_txt1, b_txt1, W_txt2, b_txt2)

    pad = ((0, NPAD - N_NODES), (0, 0))
    ego_img = jnp.pad(jnp.concatenate([img_feat, other_emb_image], axis=0), pad)
    ego_txt = jnp.pad(jnp.concatenate([txt_feat, other_emb_text], axis=0), pad)
    ego_cat = jnp.concatenate([ego_img, ego_txt], axis=0)
    rel_cat = jnp.concatenate(
        [ckg_rel_image, ckg_rel_text, ukg_rel_image, ukg_rel_text], axis=0)

    i32 = jnp.int32
    ck_head = ckg_edge_index[0].astype(i32)
    ck_tail = ckg_edge_index[1].astype(i32)
    ck_type = ckg_edge_type.astype(i32)
    uk_head = ukg_edge_index[0].astype(i32)
    uk_tail = ukg_edge_index[1].astype(i32)
    uk_type = ukg_edge_type.astype(i32)

    ck_tails2 = jnp.concatenate([ck_tail, ck_tail + NPAD])
    ck_types2 = jnp.concatenate([ck_type, ck_type + CKG_REL])
    ck_heads2 = ck_head
    uk_tails5 = jnp.concatenate([uk_tail + N_ENT, uk_tail + N_ENT + NPAD])
    uk_tails2 = jnp.concatenate([uk_tail, uk_tail + NUPAD])
    uk_types2 = jnp.concatenate([uk_type + 2 * CKG_REL,
                                 uk_type + 2 * CKG_REL + UKG_REL])
    uk_heads2 = uk_head

    degc = _deg_ckg(ck_heads2)
    cur1 = _hop1_ckg(ego_cat, rel_cat, ck_tails2, ck_types2, ck_heads2, degc)
    all_cat = _hop2_ckg(cur1, rel_cat, ck_tails2, ck_types2, ck_heads2, degc,
                        ego_cat, cur1)
    degu = _deg_ukg(uk_heads2)
    curu1 = _hop1_ukg(all_cat, rel_cat, uk_tails5, uk_types2, uk_heads2, degu)
    fu_cat = _hop2_ukg(curu1, rel_cat, uk_tails2, uk_types2, uk_heads2, degu,
                       all_cat, curu1)

    uidx = jnp.mod(user_ids.astype(i32) - N_ENT, N_USERS)
    iidx = item_ids.astype(i32)
    prod = _score_kernel(uidx, uidx + NUPAD, iidx, iidx + NPAD,
                         fu_cat, all_cat)
    return _score_reduce(prod).reshape(B)


# speedup vs baseline: 1.4153x; 1.1793x over previous
import functools

import jax
import jax.numpy as jnp
from jax import lax
from jax.experimental import pallas as pl
from jax.experimental.pallas import tpu as pltpu
from jax.experimental.pallas import tpu_sc as plsc

N_USERS = 5000
N_ITEMS = 3000
N_ENT = 5000
N_NODES = 10000
CKG_REL = 16
UKG_REL = 4
D = 128
HOPS = 2
IMG_D = 512
TXT_D = 768
B = 4096
E_CKG = 320000
E_UKG = 160000

NPAD = 10240
NUPAD = 5120
C = 80
NT = 16
ROWS_T = NPAD // NT
ROWS_U = NUPAD // NT
R = 32
ET_C = E_CKG // NT
ET_U = E_UKG // NT
NCH_C = ET_C // C
NCH_U = ET_U // C
NMOD = 2

_mesh = plsc.VectorSubcoreMesh(core_axis_name="c", subcore_axis_name="s")


def _mlp_body(x_ref, w1_ref, b1_ref, w2_ref, b2_ref, o_ref):
    h = jnp.dot(x_ref[...], w1_ref[...], preferred_element_type=jnp.float32)
    h = h + b1_ref[...]
    h = jnp.where(h >= 0, h, 0.01 * h)
    o = jnp.dot(h, w2_ref[...], preferred_element_type=jnp.float32)
    o_ref[...] = o + b2_ref[...]


def _mlp(x, w1, b1, w2, b2):
    n = x.shape[0]
    return pl.pallas_call(
        _mlp_body,
        out_shape=jax.ShapeDtypeStruct((n, D), jnp.float32),
        grid=(3,),
        in_specs=[
            pl.BlockSpec((n // 3, x.shape[1]), lambda i: (i, 0)),
            pl.BlockSpec(w1.shape, lambda i: (0, 0)),
            pl.BlockSpec(b1.shape, lambda i: (0,)),
            pl.BlockSpec(w2.shape, lambda i: (0, 0)),
            pl.BlockSpec(b2.shape, lambda i: (0,)),
        ],
        out_specs=pl.BlockSpec((n // 3, D), lambda i: (i, 0)),
    )(x, w1, b1, w2, b2)


_CONV_SCRATCH = [
    pltpu.VMEM((C,), jnp.int32),
    pltpu.VMEM((C,), jnp.int32),
    pltpu.VMEM((C,), jnp.int32),
    pltpu.VMEM((C, D), jnp.float32),
    pltpu.VMEM((C, D), jnp.float32),
    pltpu.VMEM((R, D), jnp.float32),
    pltpu.VMEM((R, D), jnp.float32),
    pltpu.VMEM((R, D), jnp.float32),
    pltpu.VMEM((R, D), jnp.float32),
    pltpu.VMEM_SHARED((NPAD, D), jnp.float32),
    pltpu.SemaphoreType.DMA,
    pltpu.SemaphoreType.DMA,
]


def _zero_acc_rows(sid, nsub, tile_rows, accb_v, acc_sh):
    zero16 = jnp.zeros((16,), jnp.float32)

    def zf(r, _):
        for k in range(8):
            accb_v[r, pl.ds(k * 16, 16)] = zero16
        return 0
    lax.fori_loop(0, R, zf, 0)

    def f(s, _):
        rb = sid * tile_rows + s * R
        pltpu.sync_copy(accb_v, acc_sh.at[pl.ds(rb, R), :])
        return 0
    lax.fori_loop(0, nsub, f, 0)


def _make_deg_kernel(out_rows, tile_rows, et, nch):
    nsub = tile_rows // R

    @functools.partial(
        pl.kernel, mesh=_mesh,
        out_type=jax.ShapeDtypeStruct((NMOD * out_rows, D), jnp.float32),
        scratch_types=_CONV_SCRATCH)
    def deg_k(heads_hbm, deg_out, tails_c, types_c, heads_c, rows_v, rel_v,
              accb_v, degb_v, egob_v, cur1b_v, acc_sh, sem_a, sem_b):
        core = lax.axis_index("c")
        sid = lax.axis_index("s")
        one16 = jnp.ones((16,), jnp.float32)

        def of(r, _):
            for k in range(8):
                rows_v[r, pl.ds(k * 16, 16)] = one16
            return 0
        lax.fori_loop(0, C, of, 0)
        _zero_acc_rows(sid, ROWS_T // R, ROWS_T, accb_v, acc_sh)
        plsc.subcore_barrier()
        eoff = sid * et

        def chunk(j, _):
            pltpu.sync_copy(heads_hbm.at[pl.ds(eoff + j * C, C)], heads_c)
            pltpu.sync_copy(rows_v, acc_sh.at[heads_c], add=True)
            return 0
        lax.fori_loop(0, nch, chunk, 0)
        plsc.subcore_barrier()
        base = core * out_rows

        def f(s, _):
            rb = sid * tile_rows + s * R
            pltpu.sync_copy(acc_sh.at[pl.ds(rb, R), :], degb_v)
            pltpu.sync_copy(degb_v, deg_out.at[pl.ds(base + rb, R), :])
            return 0
        lax.fori_loop(0, nsub, f, 0)
    return deg_k


def _make_hop_kernel(out_rows, tile_rows, e_all, et, nch, hop2, ego_off):
    nsub = tile_rows // R

    def body(src_cat, rel_cat, tails2, types2, heads1, deg_hbm, ego_hbm,
             cur1_hbm, dst_out, tails_c, types_c, heads_c, rows_v, rel_v,
             accb_v, degb_v, egob_v, cur1b_v, acc_sh, sem_a, sem_b):
        core = lax.axis_index("c")
        sid = lax.axis_index("s")
        _zero_acc_rows(sid, ROWS_T // R, ROWS_T, accb_v, acc_sh)
        plsc.subcore_barrier()
        eoff = core * e_all + sid * et

        def chunk(j, _):
            coff = eoff + j * C
            pltpu.sync_copy(tails2.at[pl.ds(coff, C)], tails_c)
            pltpu.sync_copy(types2.at[pl.ds(coff, C)], types_c)
            pltpu.sync_copy(heads1.at[pl.ds(coff - core * e_all, C)], heads_c)
            cp1 = pltpu.async_copy(src_cat.at[tails_c], rows_v, sem_a)
            cp2 = pltpu.async_copy(rel_cat.at[types_c], rel_v, sem_b)
            cp1.wait()
            cp2.wait()

            def mul(r, _):
                for k in range(8):
                    sl = pl.ds(k * 16, 16)
                    rows_v[r, sl] = rows_v[r, sl] * rel_v[r, sl]
                return 0
            lax.fori_loop(0, C, mul, 0)
            pltpu.sync_copy(rows_v, acc_sh.at[heads_c], add=True)
            return 0
        lax.fori_loop(0, nch, chunk, 0)
        plsc.subcore_barrier()
        base = core * out_rows

        def sub(s, _):
            rb = sid * tile_rows + s * R
            pltpu.sync_copy(acc_sh.at[pl.ds(rb, R), :], accb_v)
            pltpu.sync_copy(deg_hbm.at[pl.ds(base + rb, R), :], degb_v)
            if hop2:
                pltpu.sync_copy(
                    ego_hbm.at[pl.ds(core * NPAD + ego_off + rb, R), :],
                    egob_v)
                pltpu.sync_copy(cur1_hbm.at[pl.ds(base + rb, R), :], cur1b_v)

            def rowf(r, _):
                rd = 1.0 / jnp.maximum(degb_v[r, pl.ds(0, 16)], 1.0)
                for k in range(8):
                    sl = pl.ds(k * 16, 16)
                    if hop2:
                        v = (egob_v[r, sl] + cur1b_v[r, sl]
                             + accb_v[r, sl] * rd)
                        accb_v[r, sl] = v * (1.0 / 3.0)
                    else:
                        accb_v[r, sl] = accb_v[r, sl] * rd
                return 0
            lax.fori_loop(0, R, rowf, 0)
            pltpu.sync_copy(accb_v, dst_out.at[pl.ds(base + rb, R), :])
            return 0
        lax.fori_loop(0, nsub, sub, 0)

    out_t = jax.ShapeDtypeStruct((NMOD * out_rows, D), jnp.float32)
    if hop2:
        return functools.partial(pl.kernel, mesh=_mesh, out_type=out_t,
                                 scratch_types=_CONV_SCRATCH)(body)

    def body1(src_cat, rel_cat, tails2, types2, heads1, deg_hbm, dst_out,
              *scr):
        return body(src_cat, rel_cat, tails2, types2, heads1, deg_hbm,
                    None, None, dst_out, *scr)
    return functools.partial(pl.kernel, mesh=_mesh, out_type=out_t,
                             scratch_types=_CONV_SCRATCH)(body1)


_deg_ckg = _make_deg_kernel(NPAD, ROWS_T, ET_C, NCH_C)
_deg_ukg = _make_deg_kernel(NUPAD, ROWS_U, ET_U, NCH_U)
_hop1_ckg = _make_hop_kernel(NPAD, ROWS_T, E_CKG, ET_C, NCH_C, False, 0)
_hop2_ckg = _make_hop_kernel(NPAD, ROWS_T, E_CKG, ET_C, NCH_C, True, 0)
_hop1_ukg = _make_hop_kernel(NUPAD, ROWS_U, E_UKG, ET_U, NCH_U, False, 0)
_hop2_ukg = _make_hop_kernel(NUPAD, ROWS_U, E_UKG, ET_U, NCH_U, True, N_ENT)


PW = B // (2 * NT)
PCH = PW // 16


@functools.partial(
    pl.kernel,
    mesh=_mesh,
    out_type=jax.ShapeDtypeStruct((B, 2 * D), jnp.float32),
    scratch_types=[
        pltpu.VMEM((16,), jnp.int32),
        pltpu.VMEM((16,), jnp.int32),
        pltpu.VMEM((16,), jnp.int32),
        pltpu.VMEM((16,), jnp.int32),
        pltpu.VMEM((16, D), jnp.float32),
        pltpu.VMEM((16, D), jnp.float32),
        pltpu.VMEM((16, D), jnp.float32),
        pltpu.VMEM((16, D), jnp.float32),
        pltpu.VMEM((16, 2 * D), jnp.float32),
        pltpu.SemaphoreType.DMA,
    ],
)
def _score_kernel(uidx_hbm, uidxt_hbm, iidx_hbm, iidxt_hbm, fu_cat, all_cat,
                  prod_hbm, uix_img, uix_txt, iix_img, iix_txt,
                  uimg, iimg, utxt, itxt, prodb, sem):
    core = lax.axis_index("c")
    sid = lax.axis_index("s")
    wid = sid * 2 + core
    base = wid * PW

    def chunk(cj, _):
        off = base + cj * 16
        pltpu.sync_copy(uidx_hbm.at[pl.ds(off, 16)], uix_img)
        pltpu.sync_copy(uidxt_hbm.at[pl.ds(off, 16)], uix_txt)
        pltpu.sync_copy(iidx_hbm.at[pl.ds(off, 16)], iix_img)
        pltpu.sync_copy(iidxt_hbm.at[pl.ds(off, 16)], iix_txt)
        c1 = pltpu.async_copy(fu_cat.at[uix_img], uimg, sem)
        c2 = pltpu.async_copy(all_cat.at[iix_img], iimg, sem)
        c3 = pltpu.async_copy(fu_cat.at[uix_txt], utxt, sem)
        c4 = pltpu.async_copy(all_cat.at[iix_txt], itxt, sem)
        c1.wait()
        c2.wait()
        c3.wait()
        c4.wait()

        def pair(p, _):
            for k in range(8):
                sl = pl.ds(k * 16, 16)
                sl2 = pl.ds(D + k * 16, 16)
                prodb[p, sl] = uimg[p, sl] * iimg[p, sl]
                prodb[p, sl2] = utxt[p, sl] * itxt[p, sl]
            return 0
        lax.fori_loop(0, 16, pair, 0)
        pltpu.sync_copy(prodb, prod_hbm.at[pl.ds(off, 16), :])
        return 0
    lax.fori_loop(0, PCH, chunk, 0)


def _score_red_body(x_ref, o_ref):
    s = jnp.sum(x_ref[...], axis=-1)
    o_ref[...] = (1.0 / (1.0 + jnp.exp(-s))).reshape(B // D, D)


def _score_reduce(prod):
    return pl.pallas_call(
        _score_red_body,
        out_shape=jax.ShapeDtypeStruct((B // D, D), jnp.float32),
    )(prod)


def kernel(user_ids, item_ids, ckg_edge_index, ckg_edge_type,
           ukg_edge_index, ukg_edge_type, image_features, text_features,
           other_emb_image, other_emb_text, ckg_rel_image, ckg_rel_text,
           ukg_rel_image, ukg_rel_text, W_img1, b_img1, W_img2, b_img2,
           W_txt1, b_txt1, W_txt2, b_txt2):
    img_feat = _mlp(image_features, W_img1, b_img1, W_img2, b_img2)
    txt_feat = _mlp(text_features, W_txt1, b_txt1, W_txt2, b_txt2)

    pad = ((0, NPAD - N_NODES), (0, 0))
    ego_img = jnp.pad(jnp.concatenate([img_feat, other_emb_image], axis=0), pad)
    ego_txt = jnp.pad(jnp.concatenate([txt_feat, other_emb_text], axis=0), pad)
    ego_cat = jnp.concatenate([ego_img, ego_txt], axis=0)
    rel_cat = jnp.concatenate(
        [ckg_rel_image, ckg_rel_text, ukg_rel_image, ukg_rel_text], axis=0)

    i32 = jnp.int32
    ck_head = ckg_edge_index[0].astype(i32)
    ck_tail = ckg_edge_index[1].astype(i32)
    ck_type = ckg_edge_type.astype(i32)
    uk_head = ukg_edge_index[0].astype(i32)
    uk_tail = ukg_edge_index[1].astype(i32)
    uk_type = ukg_edge_type.astype(i32)

    ck_tails2 = jnp.concatenate([ck_tail, ck_tail + NPAD])
    ck_types2 = jnp.concatenate([ck_type, ck_type + CKG_REL])
    ck_heads2 = ck_head
    uk_tails5 = jnp.concatenate([uk_tail + N_ENT, uk_tail + N_ENT + NPAD])
    uk_tails2 = jnp.concatenate([uk_tail, uk_tail + NUPAD])
    uk_types2 = jnp.concatenate([uk_type + 2 * CKG_REL,
                                 uk_type + 2 * CKG_REL + UKG_REL])
    uk_heads2 = uk_head

    degc = _deg_ckg(ck_heads2)
    cur1 = _hop1_ckg(ego_cat, rel_cat, ck_tails2, ck_types2, ck_heads2, degc)
    all_cat = _hop2_ckg(cur1, rel_cat, ck_tails2, ck_types2, ck_heads2, degc,
                        ego_cat, cur1)
    degu = _deg_ukg(uk_heads2)
    curu1 = _hop1_ukg(all_cat, rel_cat, uk_tails5, uk_types2, uk_heads2, degu)
    fu_cat = _hop2_ukg(curu1, rel_cat, uk_tails2, uk_types2, uk_heads2, degu,
                       all_cat, curu1)

    uidx = jnp.mod(user_ids.astype(i32) - N_ENT, N_USERS)
    iidx = item_ids.astype(i32)
    prod = _score_kernel(uidx, uidx + NUPAD, iidx, iidx + NPAD,
                         fu_cat, all_cat)
    return _score_reduce(prod).reshape(B)
